# baseline (speedup 1.0000x reference)
import functools

import jax
import jax.numpy as jnp
from jax import lax
from jax.experimental import pallas as pl
from jax.experimental.pallas import tpu as pltpu
from jax.experimental.pallas import tpu_sc as plsc

_DIM = 256
_NQ = 4
_K = 8192
_M = 8192

_BKA = 2048


def _cbnorm_body(cb_ref, w_ref, out_ref, outb_ref):
    cb = cb_ref[0]
    w = w_ref[0]
    icb = lax.dot_general(cb, w, (((1,), (1,)), ((), ())),
                          preferred_element_type=jnp.float32)
    n = jnp.sqrt(jnp.sum(icb * icb, axis=-1, keepdims=True))
    cbn = icb / jnp.maximum(n, 1e-12)
    out_ref[0] = cbn
    outb_ref[0] = cbn.astype(jnp.bfloat16)


def _cb_norm_all(codebooks, weights):
    grid = (_NQ, _K // _BKA)
    spec = pl.BlockSpec((1, _BKA, _DIM), lambda i, k: (i, k, 0))
    return pl.pallas_call(
        _cbnorm_body,
        grid=grid,
        in_specs=[
            spec,
            pl.BlockSpec((1, _DIM, _DIM), lambda i, k: (i, 0, 0)),
        ],
        out_specs=[spec, spec],
        out_shape=[
            jax.ShapeDtypeStruct((_NQ, _K, _DIM), jnp.float32),
            jax.ShapeDtypeStruct((_NQ, _K, _DIM), jnp.bfloat16),
        ],
    )(codebooks, weights)


_BM = 1024
_KC = 2048


def _rotate_t(rT, tT):
    nx = jnp.sqrt(jnp.sum(rT * rT, axis=0, keepdims=True))
    sT = rT / jnp.maximum(nx, 1e-12)
    diff = sT - tT
    partial = jnp.sum(diff * diff)
    norm_s = jnp.sqrt(jnp.sum(sT * sT, axis=0, keepdims=True))
    norm_t = jnp.sqrt(jnp.sum(tT * tT, axis=0, keepdims=True))
    u = sT / jnp.maximum(norm_s, 1e-6)
    q = tT / jnp.maximum(norm_t, 1e-6)
    wv = u + q
    wn = jnp.sqrt(jnp.sum(wv * wv, axis=0, keepdims=True))
    w = wv / jnp.maximum(wn, 1e-6)
    sw = jnp.sum(sT * w, axis=0, keepdims=True)
    su = jnp.sum(sT * u, axis=0, keepdims=True)
    rot = sT - 2.0 * sw * w + 2.0 * su * q
    rot = rot * (norm_t / jnp.maximum(norm_s, 1e-6))
    return rot, partial


def _sim_step(k, cb_ref, xnt_ref, best_ref, besti_ref, iota_ref):
    cbc = cb_ref[...]
    s = lax.dot_general(cbc, xnt_ref[...], (((1,), (0,)), ((), ())),
                        preferred_element_type=jnp.float32)
    m = jnp.max(s, axis=0)
    a = jnp.min(jnp.where(s == m[None, :], iota_ref[...], _K), axis=0).astype(
        jnp.int32) + k * _KC
    upd = m > best_ref[...]
    best_ref[...] = jnp.where(upd, m, best_ref[...])
    besti_ref[...] = jnp.where(upd, a, besti_ref[...])


def _simargmax_body(x_ref, cb_ref, idx_ref, best_ref, besti_ref, xnt_ref,
                    iota_ref):
    mstep = pl.program_id(0)
    k = pl.program_id(1)
    nk = pl.num_programs(1)

    @pl.when((mstep == 0) & (k == 0))
    def _():
        iota_ref[...] = lax.broadcasted_iota(jnp.int32, (_KC, _BM), 0)

    @pl.when(k == 0)
    def _():
        xb = x_ref[...]
        n = jnp.sqrt(jnp.sum(xb * xb, axis=-1, keepdims=True))
        xn = xb / jnp.maximum(n, 1e-12)
        xnt_ref[...] = xn.T.astype(jnp.bfloat16)
        best_ref[...] = jnp.full((_BM,), -jnp.inf, jnp.float32)
        besti_ref[...] = jnp.zeros((_BM,), jnp.int32)

    _sim_step(k, cb_ref, xnt_ref, best_ref, besti_ref, iota_ref)

    @pl.when(k == nk - 1)
    def _():
        idx_ref[...] = besti_ref[...]


def _sim_argmax(residual, cb_bf16_i):
    grid = (_M // _BM, _K // _KC)
    return pl.pallas_call(
        _simargmax_body,
        grid=grid,
        in_specs=[
            pl.BlockSpec((_BM, _DIM), lambda m, k: (m, 0)),
            pl.BlockSpec((_KC, _DIM), lambda m, k: (k, 0)),
        ],
        out_specs=pl.BlockSpec((_BM,), lambda m, k: (m,)),
        out_shape=jax.ShapeDtypeStruct((_M,), jnp.int32),
        scratch_shapes=[
            pltpu.VMEM((_BM,), jnp.float32),
            pltpu.VMEM((_BM,), jnp.int32),
            pltpu.VMEM((_DIM, _BM), jnp.bfloat16),
            pltpu.VMEM((_KC, _BM), jnp.int32),
        ],
    )(residual, cb_bf16_i)


def _fused_body(r_ref, q_ref, cb_ref, idx_ref, res_ref, loss_ref,
                best_ref, besti_ref, xnt_ref, lacc_ref, iota_ref):
    mstep = pl.program_id(0)
    nm = pl.num_programs(0)
    k = pl.program_id(1)
    nk = pl.num_programs(1)

    @pl.when((mstep == 0) & (k == 0))
    def _():
        iota_ref[...] = lax.broadcasted_iota(jnp.int32, (_KC, _BM), 0)

    @pl.when(k == 0)
    def _():
        rT = r_ref[...].T
        tT = q_ref[...].T
        rot, partial = _rotate_t(rT, tT)
        resT = rT - rot
        res_ref[...] = resT.T
        n = jnp.sqrt(jnp.sum(resT * resT, axis=0, keepdims=True))
        xnt_ref[...] = (resT / jnp.maximum(n, 1e-12)).astype(jnp.bfloat16)
        best_ref[...] = jnp.full((_BM,), -jnp.inf, jnp.float32)
        besti_ref[...] = jnp.zeros((_BM,), jnp.int32)
        acc = jnp.where(mstep == 0, jnp.zeros((1, 1), jnp.float32),
                        lacc_ref[...]) + partial
        lacc_ref[...] = acc

        @pl.when(mstep == nm - 1)
        def _():
            loss_ref[...] = acc * (1.25 / (_M * _DIM))

    _sim_step(k, cb_ref, xnt_ref, best_ref, besti_ref, iota_ref)

    @pl.when(k == nk - 1)
    def _():
        idx_ref[...] = besti_ref[...]


def _fused_rot_sim(residual, quantized, cb_bf16_i):
    grid = (_M // _BM, _K // _KC)
    return pl.pallas_call(
        _fused_body,
        grid=grid,
        in_specs=[
            pl.BlockSpec((_BM, _DIM), lambda m, k: (m, 0)),
            pl.BlockSpec((_BM, _DIM), lambda m, k: (m, 0)),
            pl.BlockSpec((_KC, _DIM), lambda m, k: (k, 0)),
        ],
        out_specs=[
            pl.BlockSpec((_BM,), lambda m, k: (m,)),
            pl.BlockSpec((_BM, _DIM), lambda m, k: (m, 0)),
            pl.BlockSpec((1, 1), lambda m, k: (0, 0)),
        ],
        out_shape=[
            jax.ShapeDtypeStruct((_M,), jnp.int32),
            jax.ShapeDtypeStruct((_M, _DIM), jnp.float32),
            jax.ShapeDtypeStruct((1, 1), jnp.float32),
        ],
        scratch_shapes=[
            pltpu.VMEM((_BM,), jnp.float32),
            pltpu.VMEM((_BM,), jnp.int32),
            pltpu.VMEM((_DIM, _BM), jnp.bfloat16),
            pltpu.VMEM((1, 1), jnp.float32),
            pltpu.VMEM((_KC, _BM), jnp.int32),
        ],
    )(residual, quantized, cb_bf16_i)


_BMT = 2048


def _tail_body(x_ref, r_ref, q_ref, qout_ref, loss_ref, lacc_ref):
    m = pl.program_id(0)
    nm = pl.num_programs(0)
    rT = r_ref[...].T
    tT = q_ref[...].T
    rot, partial = _rotate_t(rT, tT)
    resT = rT - rot
    qout_ref[...] = x_ref[...] - resT.T
    acc = jnp.where(m == 0, jnp.zeros((1, 1), jnp.float32),
                    lacc_ref[...]) + partial
    lacc_ref[...] = acc

    @pl.when(m == nm - 1)
    def _():
        loss_ref[...] = acc * (1.25 / (_M * _DIM))


def _tail(x, residual, quantized):
    grid = (_M // _BMT,)
    spec = pl.BlockSpec((_BMT, _DIM), lambda m: (m, 0))
    return pl.pallas_call(
        _tail_body,
        grid=grid,
        in_specs=[spec, spec, spec],
        out_specs=[spec, pl.BlockSpec((1, 1), lambda m: (0, 0))],
        out_shape=[
            jax.ShapeDtypeStruct((_M, _DIM), jnp.float32),
            jax.ShapeDtypeStruct((1, 1), jnp.float32),
        ],
        scratch_shapes=[pltpu.VMEM((1, 1), jnp.float32)],
    )(x, residual, quantized)


_NW = 32
_BPW = _M // _NW


def _make_sc_gather():
    mesh = plsc.VectorSubcoreMesh(core_axis_name="c", subcore_axis_name="s")

    @functools.partial(
        pl.kernel,
        mesh=mesh,
        out_type=jax.ShapeDtypeStruct((_M, _DIM), jnp.float32),
        scratch_types=[
            pltpu.VMEM((_BPW,), jnp.int32),
            pltpu.VMEM((_BPW, _DIM), jnp.float32),
            pltpu.SemaphoreType.DMA,
        ],
    )
    def gather(table_hbm, idx_hbm, out_hbm, idx_v, rows_v, sem):
        wid = lax.axis_index("s") * 2 + lax.axis_index("c")
        base = wid * _BPW
        pltpu.sync_copy(idx_hbm.at[pl.ds(base, _BPW)], idx_v)
        pltpu.async_copy(table_hbm.at[idx_v], rows_v, sem).wait()
        pltpu.sync_copy(rows_v, out_hbm.at[pl.ds(base, _BPW)])

    return gather


_sc_gather = _make_sc_gather()


def kernel(x, codebooks, weights):
    b, n, d = x.shape
    xf = x.reshape(_M, _DIM)
    cb_norm, cb_bf16 = _cb_norm_all(codebooks, weights)

    idx = _sim_argmax(xf, cb_bf16[0])
    quantized = _sc_gather(cb_norm[0], idx)
    residual = xf
    all_idx = [idx]
    all_loss = []
    for i in range(1, _NQ):
        idx, residual, loss = _fused_rot_sim(residual, quantized, cb_bf16[i])
        quantized = _sc_gather(cb_norm[i], idx)
        all_idx.append(idx)
        all_loss.append(loss.reshape(()))
    qout, loss = _tail(xf, residual, quantized)
    all_loss.append(loss.reshape(()))

    quantized_out = qout.reshape(b, n, d)
    indices = jnp.stack(all_idx, axis=-1).reshape(b, n, _NQ)
    losses = jnp.stack(all_loss, axis=-1)
    return quantized_out, indices, losses


# speedup vs baseline: 1.0274x; 1.0274x over previous
import functools

import jax
import jax.numpy as jnp
from jax import lax
from jax.experimental import pallas as pl
from jax.experimental.pallas import tpu as pltpu
from jax.experimental.pallas import tpu_sc as plsc

_DIM = 256
_NQ = 4
_K = 8192
_M = 8192

_BKA = 2048


def _cbnorm_body(cb_ref, w_ref, out_ref, outb_ref):
    cb = cb_ref[0]
    w = w_ref[0]
    icb = lax.dot_general(cb, w, (((1,), (1,)), ((), ())),
                          preferred_element_type=jnp.float32)
    n = jnp.sqrt(jnp.sum(icb * icb, axis=-1, keepdims=True))
    cbn = icb / jnp.maximum(n, 1e-12)
    out_ref[0] = cbn
    outb_ref[0] = cbn.astype(jnp.bfloat16)


def _cb_norm_all(codebooks, weights):
    grid = (_NQ, _K // _BKA)
    spec = pl.BlockSpec((1, _BKA, _DIM), lambda i, k: (i, k, 0))
    return pl.pallas_call(
        _cbnorm_body,
        grid=grid,
        in_specs=[
            spec,
            pl.BlockSpec((1, _DIM, _DIM), lambda i, k: (i, 0, 0)),
        ],
        out_specs=[spec, spec],
        out_shape=[
            jax.ShapeDtypeStruct((_NQ, _K, _DIM), jnp.float32),
            jax.ShapeDtypeStruct((_NQ, _K, _DIM), jnp.bfloat16),
        ],
    )(codebooks, weights)


_BM = 1024
_KC = 2048


def _rotate_t(rT, tT):
    nx = jnp.sqrt(jnp.sum(rT * rT, axis=0, keepdims=True))
    sT = rT / jnp.maximum(nx, 1e-12)
    diff = sT - tT
    partial = jnp.sum(diff * diff)
    norm_s = jnp.sqrt(jnp.sum(sT * sT, axis=0, keepdims=True))
    norm_t = jnp.sqrt(jnp.sum(tT * tT, axis=0, keepdims=True))
    u = sT / jnp.maximum(norm_s, 1e-6)
    q = tT / jnp.maximum(norm_t, 1e-6)
    wv = u + q
    wn = jnp.sqrt(jnp.sum(wv * wv, axis=0, keepdims=True))
    w = wv / jnp.maximum(wn, 1e-6)
    sw = jnp.sum(sT * w, axis=0, keepdims=True)
    su = jnp.sum(sT * u, axis=0, keepdims=True)
    rot = sT - 2.0 * sw * w + 2.0 * su * q
    rot = rot * (norm_t / jnp.maximum(norm_s, 1e-6))
    return rot, partial


def _sim_step(k, cb_ref, xnt_ref, best_ref, besti_ref):
    cbc = cb_ref[...]
    s = lax.dot_general(cbc, xnt_ref[...], (((1,), (0,)), ((), ())),
                        preferred_element_type=jnp.float32)
    m = jnp.max(s, axis=0)
    iota = lax.broadcasted_iota(jnp.int32, (_KC, _BM), 0)
    a = jnp.min(jnp.where(s == m[None, :], iota, _K), axis=0).astype(
        jnp.int32) + k * _KC
    upd = m > best_ref[...]
    best_ref[...] = jnp.where(upd, m, best_ref[...])
    besti_ref[...] = jnp.where(upd, a, besti_ref[...])


def _simargmax_body(x_ref, cb_ref, idx_ref, best_ref, besti_ref, xnt_ref):
    k = pl.program_id(1)
    nk = pl.num_programs(1)

    @pl.when(k == 0)
    def _():
        xb = x_ref[...]
        n = jnp.sqrt(jnp.sum(xb * xb, axis=-1, keepdims=True))
        xn = xb / jnp.maximum(n, 1e-12)
        xnt_ref[...] = xn.T.astype(jnp.bfloat16)
        best_ref[...] = jnp.full((_BM,), -jnp.inf, jnp.float32)
        besti_ref[...] = jnp.zeros((_BM,), jnp.int32)

    _sim_step(k, cb_ref, xnt_ref, best_ref, besti_ref)

    @pl.when(k == nk - 1)
    def _():
        idx_ref[...] = besti_ref[...]


def _sim_argmax(residual, cb_bf16_i):
    grid = (_M // _BM, _K // _KC)
    return pl.pallas_call(
        _simargmax_body,
        grid=grid,
        in_specs=[
            pl.BlockSpec((_BM, _DIM), lambda m, k: (m, 0)),
            pl.BlockSpec((_KC, _DIM), lambda m, k: (k, 0)),
        ],
        out_specs=pl.BlockSpec((_BM,), lambda m, k: (m,)),
        out_shape=jax.ShapeDtypeStruct((_M,), jnp.int32),
        scratch_shapes=[
            pltpu.VMEM((_BM,), jnp.float32),
            pltpu.VMEM((_BM,), jnp.int32),
            pltpu.VMEM((_DIM, _BM), jnp.bfloat16),
        ],
    )(residual, cb_bf16_i)


def _fused_body(r_ref, q_ref, cb_ref, idx_ref, res_ref, loss_ref,
                best_ref, besti_ref, xnt_ref, lacc_ref):
    mstep = pl.program_id(0)
    nm = pl.num_programs(0)
    k = pl.program_id(1)
    nk = pl.num_programs(1)

    @pl.when(k == 0)
    def _():
        rT = r_ref[...].T
        tT = q_ref[...].T
        rot, partial = _rotate_t(rT, tT)
        resT = rT - rot
        res_ref[...] = resT.T
        n = jnp.sqrt(jnp.sum(resT * resT, axis=0, keepdims=True))
        xnt_ref[...] = (resT / jnp.maximum(n, 1e-12)).astype(jnp.bfloat16)
        best_ref[...] = jnp.full((_BM,), -jnp.inf, jnp.float32)
        besti_ref[...] = jnp.zeros((_BM,), jnp.int32)
        acc = jnp.where(mstep == 0, jnp.zeros((1, 1), jnp.float32),
                        lacc_ref[...]) + partial
        lacc_ref[...] = acc

        @pl.when(mstep == nm - 1)
        def _():
            loss_ref[...] = acc * (1.25 / (_M * _DIM))

    _sim_step(k, cb_ref, xnt_ref, best_ref, besti_ref)

    @pl.when(k == nk - 1)
    def _():
        idx_ref[...] = besti_ref[...]


def _fused_rot_sim(residual, quantized, cb_bf16_i):
    grid = (_M // _BM, _K // _KC)
    return pl.pallas_call(
        _fused_body,
        grid=grid,
        in_specs=[
            pl.BlockSpec((_BM, _DIM), lambda m, k: (m, 0)),
            pl.BlockSpec((_BM, _DIM), lambda m, k: (m, 0)),
            pl.BlockSpec((_KC, _DIM), lambda m, k: (k, 0)),
        ],
        out_specs=[
            pl.BlockSpec((_BM,), lambda m, k: (m,)),
            pl.BlockSpec((_BM, _DIM), lambda m, k: (m, 0)),
            pl.BlockSpec((1, 1), lambda m, k: (0, 0)),
        ],
        out_shape=[
            jax.ShapeDtypeStruct((_M,), jnp.int32),
            jax.ShapeDtypeStruct((_M, _DIM), jnp.float32),
            jax.ShapeDtypeStruct((1, 1), jnp.float32),
        ],
        scratch_shapes=[
            pltpu.VMEM((_BM,), jnp.float32),
            pltpu.VMEM((_BM,), jnp.int32),
            pltpu.VMEM((_DIM, _BM), jnp.bfloat16),
            pltpu.VMEM((1, 1), jnp.float32),
        ],
    )(residual, quantized, cb_bf16_i)


_BMT = 2048


def _tail_body(x_ref, r_ref, q_ref, qout_ref, loss_ref, lacc_ref):
    m = pl.program_id(0)
    nm = pl.num_programs(0)
    rT = r_ref[...].T
    tT = q_ref[...].T
    rot, partial = _rotate_t(rT, tT)
    resT = rT - rot
    qout_ref[...] = x_ref[...] - resT.T
    acc = jnp.where(m == 0, jnp.zeros((1, 1), jnp.float32),
                    lacc_ref[...]) + partial
    lacc_ref[...] = acc

    @pl.when(m == nm - 1)
    def _():
        loss_ref[...] = acc * (1.25 / (_M * _DIM))


def _tail(x, residual, quantized):
    grid = (_M // _BMT,)
    spec = pl.BlockSpec((_BMT, _DIM), lambda m: (m, 0))
    return pl.pallas_call(
        _tail_body,
        grid=grid,
        in_specs=[spec, spec, spec],
        out_specs=[spec, pl.BlockSpec((1, 1), lambda m: (0, 0))],
        out_shape=[
            jax.ShapeDtypeStruct((_M, _DIM), jnp.float32),
            jax.ShapeDtypeStruct((1, 1), jnp.float32),
        ],
        scratch_shapes=[pltpu.VMEM((1, 1), jnp.float32)],
    )(x, residual, quantized)


_NW = 32
_BPW = _M // _NW


def _make_sc_gather():
    mesh = plsc.VectorSubcoreMesh(core_axis_name="c", subcore_axis_name="s")

    @functools.partial(
        pl.kernel,
        mesh=mesh,
        out_type=jax.ShapeDtypeStruct((_M, _DIM), jnp.float32),
        scratch_types=[
            pltpu.VMEM((_BPW,), jnp.int32),
            pltpu.VMEM((_BPW, _DIM), jnp.float32),
            pltpu.SemaphoreType.DMA,
        ],
    )
    def gather(table_hbm, idx_hbm, out_hbm, idx_v, rows_v, sem):
        wid = lax.axis_index("s") * 2 + lax.axis_index("c")
        base = wid * _BPW
        pltpu.sync_copy(idx_hbm.at[pl.ds(base, _BPW)], idx_v)
        pltpu.async_copy(table_hbm.at[idx_v], rows_v, sem).wait()
        pltpu.sync_copy(rows_v, out_hbm.at[pl.ds(base, _BPW)])

    return gather


_sc_gather = _make_sc_gather()


def kernel(x, codebooks, weights):
    b, n, d = x.shape
    xf = x.reshape(_M, _DIM)
    cb_norm, cb_bf16 = _cb_norm_all(codebooks, weights)

    idx = _sim_argmax(xf, cb_bf16[0])
    quantized = _sc_gather(cb_norm[0], idx)
    residual = xf
    all_idx = [idx]
    all_loss = []
    for i in range(1, _NQ):
        idx, residual, loss = _fused_rot_sim(residual, quantized, cb_bf16[i])
        quantized = _sc_gather(cb_norm[i], idx)
        all_idx.append(idx)
        all_loss.append(loss.reshape(()))
    qout, loss = _tail(xf, residual, quantized)
    all_loss.append(loss.reshape(()))

    quantized_out = qout.reshape(b, n, d)
    indices = jnp.stack(all_idx, axis=-1).reshape(b, n, _NQ)
    losses = jnp.stack(all_loss, axis=-1)
    return quantized_out, indices, losses


# speedup vs baseline: 1.4356x; 1.3973x over previous
import functools

import jax
import jax.numpy as jnp
from jax import lax
from jax.experimental import pallas as pl
from jax.experimental.pallas import tpu as pltpu
from jax.experimental.pallas import tpu_sc as plsc

_DIM = 256
_NQ = 4
_K = 8192
_M = 8192

_BKA = 2048


def _cbnorm_body(cb_ref, w_ref, out_ref, outb_ref):
    cb = cb_ref[0]
    w = w_ref[0]
    icb = lax.dot_general(cb, w, (((1,), (1,)), ((), ())),
                          preferred_element_type=jnp.float32)
    n = jnp.sqrt(jnp.sum(icb * icb, axis=-1, keepdims=True))
    cbn = icb / jnp.maximum(n, 1e-12)
    out_ref[0] = cbn
    outb_ref[0] = cbn.astype(jnp.bfloat16)


def _cb_norm_all(codebooks, weights):
    grid = (_NQ, _K // _BKA)
    spec = pl.BlockSpec((1, _BKA, _DIM), lambda i, k: (i, k, 0))
    return pl.pallas_call(
        _cbnorm_body,
        grid=grid,
        in_specs=[
            spec,
            pl.BlockSpec((1, _DIM, _DIM), lambda i, k: (i, 0, 0)),
        ],
        out_specs=[spec, spec],
        out_shape=[
            jax.ShapeDtypeStruct((_NQ, _K, _DIM), jnp.float32),
            jax.ShapeDtypeStruct((_NQ, _K, _DIM), jnp.bfloat16),
        ],
    )(codebooks, weights)


_BM = 1024
_KC = 2048


def _rotate_t(rT, tT):
    nx = jnp.sqrt(jnp.sum(rT * rT, axis=0, keepdims=True))
    sT = rT / jnp.maximum(nx, 1e-12)
    diff = sT - tT
    partial = jnp.sum(diff * diff)
    norm_s = jnp.sqrt(jnp.sum(sT * sT, axis=0, keepdims=True))
    norm_t = jnp.sqrt(jnp.sum(tT * tT, axis=0, keepdims=True))
    u = sT / jnp.maximum(norm_s, 1e-6)
    q = tT / jnp.maximum(norm_t, 1e-6)
    wv = u + q
    wn = jnp.sqrt(jnp.sum(wv * wv, axis=0, keepdims=True))
    w = wv / jnp.maximum(wn, 1e-6)
    sw = jnp.sum(sT * w, axis=0, keepdims=True)
    su = jnp.sum(sT * u, axis=0, keepdims=True)
    rot = sT - 2.0 * sw * w + 2.0 * su * q
    rot = rot * (norm_t / jnp.maximum(norm_s, 1e-6))
    return rot, partial


def _sim_step(k, nk, cb_ref, xnt_ref, best8_ref, bestr_ref, idx_ref):
    cbc = cb_ref[...]
    s = lax.dot_general(cbc, xnt_ref[...], (((1,), (0,)), ((), ())),
                        preferred_element_type=jnp.float32)
    best8 = best8_ref[...]
    bestr = bestr_ref[...]
    for r in range(_KC // 8):
        row = lax.slice_in_dim(s, 8 * r, 8 * (r + 1), axis=0)
        upd = row > best8
        best8 = jnp.where(upd, row, best8)
        bestr = jnp.where(upd, k * (_KC // 8) + r, bestr)
    best8_ref[...] = best8
    bestr_ref[...] = bestr

    @pl.when(k == nk - 1)
    def _():
        m = jnp.max(best8, axis=0)
        sub = lax.broadcasted_iota(jnp.int32, (8, _BM), 0)
        code8 = bestr * 8 + sub
        idx_ref[...] = jnp.min(
            jnp.where(best8 == m[None, :], code8, _NQ * _K), axis=0)


def _simargmax_body(x_ref, cb_ref, idx_ref, best8_ref, bestr_ref, xnt_ref):
    k = pl.program_id(1)
    nk = pl.num_programs(1)

    @pl.when(k == 0)
    def _():
        xb = x_ref[...]
        n = jnp.sqrt(jnp.sum(xb * xb, axis=-1, keepdims=True))
        xn = xb / jnp.maximum(n, 1e-12)
        xnt_ref[...] = xn.T.astype(jnp.bfloat16)
        best8_ref[...] = jnp.full((8, _BM), -jnp.inf, jnp.float32)
        bestr_ref[...] = jnp.zeros((8, _BM), jnp.int32)

    _sim_step(k, nk, cb_ref, xnt_ref, best8_ref, bestr_ref, idx_ref)


def _sim_argmax(residual, cb_bf16_i):
    grid = (_M // _BM, _K // _KC)
    return pl.pallas_call(
        _simargmax_body,
        grid=grid,
        in_specs=[
            pl.BlockSpec((_BM, _DIM), lambda m, k: (m, 0)),
            pl.BlockSpec((_KC, _DIM), lambda m, k: (k, 0)),
        ],
        out_specs=pl.BlockSpec((_BM,), lambda m, k: (m,)),
        out_shape=jax.ShapeDtypeStruct((_M,), jnp.int32),
        scratch_shapes=[
            pltpu.VMEM((8, _BM), jnp.float32),
            pltpu.VMEM((8, _BM), jnp.int32),
            pltpu.VMEM((_DIM, _BM), jnp.bfloat16),
        ],
    )(residual, cb_bf16_i)


def _fused_body(r_ref, q_ref, cb_ref, idx_ref, res_ref, loss_ref,
                best8_ref, bestr_ref, xnt_ref, lacc_ref):
    mstep = pl.program_id(0)
    nm = pl.num_programs(0)
    k = pl.program_id(1)
    nk = pl.num_programs(1)

    @pl.when(k == 0)
    def _():
        rT = r_ref[...].T
        tT = q_ref[...].T
        rot, partial = _rotate_t(rT, tT)
        resT = rT - rot
        res_ref[...] = resT.T
        n = jnp.sqrt(jnp.sum(resT * resT, axis=0, keepdims=True))
        xnt_ref[...] = (resT / jnp.maximum(n, 1e-12)).astype(jnp.bfloat16)
        best8_ref[...] = jnp.full((8, _BM), -jnp.inf, jnp.float32)
        bestr_ref[...] = jnp.zeros((8, _BM), jnp.int32)
        acc = jnp.where(mstep == 0, jnp.zeros((1, 1), jnp.float32),
                        lacc_ref[...]) + partial
        lacc_ref[...] = acc

        @pl.when(mstep == nm - 1)
        def _():
            loss_ref[...] = acc * (1.25 / (_M * _DIM))

    _sim_step(k, nk, cb_ref, xnt_ref, best8_ref, bestr_ref, idx_ref)


def _fused_rot_sim(residual, quantized, cb_bf16_i):
    grid = (_M // _BM, _K // _KC)
    return pl.pallas_call(
        _fused_body,
        grid=grid,
        in_specs=[
            pl.BlockSpec((_BM, _DIM), lambda m, k: (m, 0)),
            pl.BlockSpec((_BM, _DIM), lambda m, k: (m, 0)),
            pl.BlockSpec((_KC, _DIM), lambda m, k: (k, 0)),
        ],
        out_specs=[
            pl.BlockSpec((_BM,), lambda m, k: (m,)),
            pl.BlockSpec((_BM, _DIM), lambda m, k: (m, 0)),
            pl.BlockSpec((1, 1), lambda m, k: (0, 0)),
        ],
        out_shape=[
            jax.ShapeDtypeStruct((_M,), jnp.int32),
            jax.ShapeDtypeStruct((_M, _DIM), jnp.float32),
            jax.ShapeDtypeStruct((1, 1), jnp.float32),
        ],
        scratch_shapes=[
            pltpu.VMEM((8, _BM), jnp.float32),
            pltpu.VMEM((8, _BM), jnp.int32),
            pltpu.VMEM((_DIM, _BM), jnp.bfloat16),
            pltpu.VMEM((1, 1), jnp.float32),
        ],
    )(residual, quantized, cb_bf16_i)


_BMT = 2048


def _tail_body(x_ref, r_ref, q_ref, qout_ref, loss_ref, lacc_ref):
    m = pl.program_id(0)
    nm = pl.num_programs(0)
    rT = r_ref[...].T
    tT = q_ref[...].T
    rot, partial = _rotate_t(rT, tT)
    resT = rT - rot
    qout_ref[...] = x_ref[...] - resT.T
    acc = jnp.where(m == 0, jnp.zeros((1, 1), jnp.float32),
                    lacc_ref[...]) + partial
    lacc_ref[...] = acc

    @pl.when(m == nm - 1)
    def _():
        loss_ref[...] = acc * (1.25 / (_M * _DIM))


def _tail(x, residual, quantized):
    grid = (_M // _BMT,)
    spec = pl.BlockSpec((_BMT, _DIM), lambda m: (m, 0))
    return pl.pallas_call(
        _tail_body,
        grid=grid,
        in_specs=[spec, spec, spec],
        out_specs=[spec, pl.BlockSpec((1, 1), lambda m: (0, 0))],
        out_shape=[
            jax.ShapeDtypeStruct((_M, _DIM), jnp.float32),
            jax.ShapeDtypeStruct((1, 1), jnp.float32),
        ],
        scratch_shapes=[pltpu.VMEM((1, 1), jnp.float32)],
    )(x, residual, quantized)


_NW = 32
_BPW = _M // _NW


def _make_sc_gather():
    mesh = plsc.VectorSubcoreMesh(core_axis_name="c", subcore_axis_name="s")

    @functools.partial(
        pl.kernel,
        mesh=mesh,
        out_type=jax.ShapeDtypeStruct((_M, _DIM), jnp.float32),
        scratch_types=[
            pltpu.VMEM((_BPW,), jnp.int32),
            pltpu.VMEM((_BPW, _DIM), jnp.float32),
            pltpu.SemaphoreType.DMA,
        ],
    )
    def gather(table_hbm, idx_hbm, out_hbm, idx_v, rows_v, sem):
        wid = lax.axis_index("s") * 2 + lax.axis_index("c")
        base = wid * _BPW
        pltpu.sync_copy(idx_hbm.at[pl.ds(base, _BPW)], idx_v)
        pltpu.async_copy(table_hbm.at[idx_v], rows_v, sem).wait()
        pltpu.sync_copy(rows_v, out_hbm.at[pl.ds(base, _BPW)])

    return gather


_sc_gather = _make_sc_gather()


def kernel(x, codebooks, weights):
    b, n, d = x.shape
    xf = x.reshape(_M, _DIM)
    cb_norm, cb_bf16 = _cb_norm_all(codebooks, weights)

    idx = _sim_argmax(xf, cb_bf16[0])
    quantized = _sc_gather(cb_norm[0], idx)
    residual = xf
    all_idx = [idx]
    all_loss = []
    for i in range(1, _NQ):
        idx, residual, loss = _fused_rot_sim(residual, quantized, cb_bf16[i])
        quantized = _sc_gather(cb_norm[i], idx)
        all_idx.append(idx)
        all_loss.append(loss.reshape(()))
    qout, loss = _tail(xf, residual, quantized)
    all_loss.append(loss.reshape(()))

    quantized_out = qout.reshape(b, n, d)
    indices = jnp.stack(all_idx, axis=-1).reshape(b, n, _NQ)
    losses = jnp.stack(all_loss, axis=-1)
    return quantized_out, indices, losses


# speedup vs baseline: 1.5405x; 1.0731x over previous
import functools

import jax
import jax.numpy as jnp
from jax import lax
from jax.experimental import pallas as pl
from jax.experimental.pallas import tpu as pltpu
from jax.experimental.pallas import tpu_sc as plsc

_DIM = 256
_NQ = 4
_K = 8192
_M = 8192

_BKA = 2048


def _cbnorm_body(cb_ref, w_ref, out_ref, outb_ref):
    cb = cb_ref[0]
    w = w_ref[0]
    icb = lax.dot_general(cb, w, (((1,), (1,)), ((), ())),
                          preferred_element_type=jnp.float32)
    n = jnp.sqrt(jnp.sum(icb * icb, axis=-1, keepdims=True))
    cbn = icb / jnp.maximum(n, 1e-12)
    out_ref[0] = cbn
    outb_ref[0] = cbn.astype(jnp.bfloat16)


def _cb_norm_all(codebooks, weights):
    grid = (_NQ, _K // _BKA)
    spec = pl.BlockSpec((1, _BKA, _DIM), lambda i, k: (i, k, 0))
    return pl.pallas_call(
        _cbnorm_body,
        grid=grid,
        in_specs=[
            spec,
            pl.BlockSpec((1, _DIM, _DIM), lambda i, k: (i, 0, 0)),
        ],
        out_specs=[spec, spec],
        out_shape=[
            jax.ShapeDtypeStruct((_NQ, _K, _DIM), jnp.float32),
            jax.ShapeDtypeStruct((_NQ, _K, _DIM), jnp.bfloat16),
        ],
    )(codebooks, weights)


_BM = 2048
_KC = 2048


def _rotate_t(rT, tT):
    nx = jnp.sqrt(jnp.sum(rT * rT, axis=0, keepdims=True))
    sT = rT / jnp.maximum(nx, 1e-12)
    diff = sT - tT
    partial = jnp.sum(diff * diff)
    norm_s = jnp.sqrt(jnp.sum(sT * sT, axis=0, keepdims=True))
    norm_t = jnp.sqrt(jnp.sum(tT * tT, axis=0, keepdims=True))
    u = sT / jnp.maximum(norm_s, 1e-6)
    q = tT / jnp.maximum(norm_t, 1e-6)
    wv = u + q
    wn = jnp.sqrt(jnp.sum(wv * wv, axis=0, keepdims=True))
    w = wv / jnp.maximum(wn, 1e-6)
    sw = jnp.sum(sT * w, axis=0, keepdims=True)
    su = jnp.sum(sT * u, axis=0, keepdims=True)
    rot = sT - 2.0 * sw * w + 2.0 * su * q
    rot = rot * (norm_t / jnp.maximum(norm_s, 1e-6))
    return rot, partial


def _sim_step(k, nk, cb_ref, xnt_ref, best8_ref, bestr_ref, idx_ref):
    cbc = cb_ref[...]
    s = lax.dot_general(cbc, xnt_ref[...], (((1,), (0,)), ((), ())),
                        preferred_element_type=jnp.float32)
    best8 = best8_ref[...]
    bestr = bestr_ref[...]
    for r in range(_KC // 8):
        row = lax.slice_in_dim(s, 8 * r, 8 * (r + 1), axis=0)
        upd = row > best8
        best8 = jnp.where(upd, row, best8)
        bestr = jnp.where(upd, k * (_KC // 8) + r, bestr)
    best8_ref[...] = best8
    bestr_ref[...] = bestr

    @pl.when(k == nk - 1)
    def _():
        m = jnp.max(best8, axis=0)
        sub = lax.broadcasted_iota(jnp.int32, (8, _BM), 0)
        code8 = bestr * 8 + sub
        idx_ref[...] = jnp.min(
            jnp.where(best8 == m[None, :], code8, _NQ * _K), axis=0)


def _simargmax_body(x_ref, cb_ref, idx_ref, best8_ref, bestr_ref, xnt_ref):
    k = pl.program_id(1)
    nk = pl.num_programs(1)

    @pl.when(k == 0)
    def _():
        xb = x_ref[...]
        n = jnp.sqrt(jnp.sum(xb * xb, axis=-1, keepdims=True))
        xn = xb / jnp.maximum(n, 1e-12)
        xnt_ref[...] = xn.T.astype(jnp.bfloat16)
        best8_ref[...] = jnp.full((8, _BM), -jnp.inf, jnp.float32)
        bestr_ref[...] = jnp.zeros((8, _BM), jnp.int32)

    _sim_step(k, nk, cb_ref, xnt_ref, best8_ref, bestr_ref, idx_ref)


def _sim_argmax(residual, cb_bf16_i):
    grid = (_M // _BM, _K // _KC)
    return pl.pallas_call(
        _simargmax_body,
        grid=grid,
        in_specs=[
            pl.BlockSpec((_BM, _DIM), lambda m, k: (m, 0)),
            pl.BlockSpec((_KC, _DIM), lambda m, k: (k, 0)),
        ],
        out_specs=pl.BlockSpec((_BM,), lambda m, k: (m,)),
        out_shape=jax.ShapeDtypeStruct((_M,), jnp.int32),
        scratch_shapes=[
            pltpu.VMEM((8, _BM), jnp.float32),
            pltpu.VMEM((8, _BM), jnp.int32),
            pltpu.VMEM((_DIM, _BM), jnp.bfloat16),
        ],
    )(residual, cb_bf16_i)


def _fused_body(r_ref, q_ref, cb_ref, idx_ref, res_ref, loss_ref,
                best8_ref, bestr_ref, xnt_ref, lacc_ref):
    mstep = pl.program_id(0)
    nm = pl.num_programs(0)
    k = pl.program_id(1)
    nk = pl.num_programs(1)

    @pl.when(k == 0)
    def _():
        rT = r_ref[...].T
        tT = q_ref[...].T
        rot, partial = _rotate_t(rT, tT)
        resT = rT - rot
        res_ref[...] = resT.T
        n = jnp.sqrt(jnp.sum(resT * resT, axis=0, keepdims=True))
        xnt_ref[...] = (resT / jnp.maximum(n, 1e-12)).astype(jnp.bfloat16)
        best8_ref[...] = jnp.full((8, _BM), -jnp.inf, jnp.float32)
        bestr_ref[...] = jnp.zeros((8, _BM), jnp.int32)
        acc = jnp.where(mstep == 0, jnp.zeros((1, 1), jnp.float32),
                        lacc_ref[...]) + partial
        lacc_ref[...] = acc

        @pl.when(mstep == nm - 1)
        def _():
            loss_ref[...] = acc * (1.25 / (_M * _DIM))

    _sim_step(k, nk, cb_ref, xnt_ref, best8_ref, bestr_ref, idx_ref)


def _fused_rot_sim(residual, quantized, cb_bf16_i):
    grid = (_M // _BM, _K // _KC)
    return pl.pallas_call(
        _fused_body,
        grid=grid,
        in_specs=[
            pl.BlockSpec((_BM, _DIM), lambda m, k: (m, 0)),
            pl.BlockSpec((_BM, _DIM), lambda m, k: (m, 0)),
            pl.BlockSpec((_KC, _DIM), lambda m, k: (k, 0)),
        ],
        out_specs=[
            pl.BlockSpec((_BM,), lambda m, k: (m,)),
            pl.BlockSpec((_BM, _DIM), lambda m, k: (m, 0)),
            pl.BlockSpec((1, 1), lambda m, k: (0, 0)),
        ],
        out_shape=[
            jax.ShapeDtypeStruct((_M,), jnp.int32),
            jax.ShapeDtypeStruct((_M, _DIM), jnp.float32),
            jax.ShapeDtypeStruct((1, 1), jnp.float32),
        ],
        scratch_shapes=[
            pltpu.VMEM((8, _BM), jnp.float32),
            pltpu.VMEM((8, _BM), jnp.int32),
            pltpu.VMEM((_DIM, _BM), jnp.bfloat16),
            pltpu.VMEM((1, 1), jnp.float32),
        ],
    )(residual, quantized, cb_bf16_i)


_BMT = 2048


def _tail_body(x_ref, r_ref, q_ref, qout_ref, loss_ref, lacc_ref):
    m = pl.program_id(0)
    nm = pl.num_programs(0)
    rT = r_ref[...].T
    tT = q_ref[...].T
    rot, partial = _rotate_t(rT, tT)
    resT = rT - rot
    qout_ref[...] = x_ref[...] - resT.T
    acc = jnp.where(m == 0, jnp.zeros((1, 1), jnp.float32),
                    lacc_ref[...]) + partial
    lacc_ref[...] = acc

    @pl.when(m == nm - 1)
    def _():
        loss_ref[...] = acc * (1.25 / (_M * _DIM))


def _tail(x, residual, quantized):
    grid = (_M // _BMT,)
    spec = pl.BlockSpec((_BMT, _DIM), lambda m: (m, 0))
    return pl.pallas_call(
        _tail_body,
        grid=grid,
        in_specs=[spec, spec, spec],
        out_specs=[spec, pl.BlockSpec((1, 1), lambda m: (0, 0))],
        out_shape=[
            jax.ShapeDtypeStruct((_M, _DIM), jnp.float32),
            jax.ShapeDtypeStruct((1, 1), jnp.float32),
        ],
        scratch_shapes=[pltpu.VMEM((1, 1), jnp.float32)],
    )(x, residual, quantized)


_NW = 32
_BPW = _M // _NW


def _make_sc_gather():
    mesh = plsc.VectorSubcoreMesh(core_axis_name="c", subcore_axis_name="s")

    @functools.partial(
        pl.kernel,
        mesh=mesh,
        out_type=jax.ShapeDtypeStruct((_M, _DIM), jnp.float32),
        scratch_types=[
            pltpu.VMEM((_BPW,), jnp.int32),
            pltpu.VMEM((_BPW, _DIM), jnp.float32),
            pltpu.SemaphoreType.DMA,
        ],
    )
    def gather(table_hbm, idx_hbm, out_hbm, idx_v, rows_v, sem):
        wid = lax.axis_index("s") * 2 + lax.axis_index("c")
        base = wid * _BPW
        pltpu.sync_copy(idx_hbm.at[pl.ds(base, _BPW)], idx_v)
        pltpu.async_copy(table_hbm.at[idx_v], rows_v, sem).wait()
        pltpu.sync_copy(rows_v, out_hbm.at[pl.ds(base, _BPW)])

    return gather


_sc_gather = _make_sc_gather()


def kernel(x, codebooks, weights):
    b, n, d = x.shape
    xf = x.reshape(_M, _DIM)
    cb_norm, cb_bf16 = _cb_norm_all(codebooks, weights)

    idx = _sim_argmax(xf, cb_bf16[0])
    quantized = _sc_gather(cb_norm[0], idx)
    residual = xf
    all_idx = [idx]
    all_loss = []
    for i in range(1, _NQ):
        idx, residual, loss = _fused_rot_sim(residual, quantized, cb_bf16[i])
        quantized = _sc_gather(cb_norm[i], idx)
        all_idx.append(idx)
        all_loss.append(loss.reshape(()))
    qout, loss = _tail(xf, residual, quantized)
    all_loss.append(loss.reshape(()))

    quantized_out = qout.reshape(b, n, d)
    indices = jnp.stack(all_idx, axis=-1).reshape(b, n, _NQ)
    losses = jnp.stack(all_loss, axis=-1)
    return quantized_out, indices, losses
